# baseline (speedup 1.0000x reference)
import functools

import jax
import jax.numpy as jnp
from jax import lax
from jax.experimental import pallas as pl
from jax.experimental.pallas import tpu as pltpu
from jax.experimental.pallas import tpu_sc as plsc

BATCH = 4096
SEQ = 50
DIM = 64
B = BATCH * SEQ
NC, NS = 2, 16
NW = NC * NS
BPW = B // NW
G = 128
NG = BPW // G


def _gather_body(idx_hbm, table_hbm, out_hbm, idx_v, rows_v, sem):
    wid = lax.axis_index("s") * NC + lax.axis_index("c")
    base = wid * BPW
    pltpu.sync_copy(idx_hbm.at[wid], idx_v)

    def step(j, _):
        pltpu.async_copy(table_hbm.at[idx_v.at[j]], rows_v, sem).wait()
        pltpu.sync_copy(rows_v, out_hbm.at[pl.ds(base + j * G, G)])
        return ()

    lax.fori_loop(0, NG, step, ())


@jax.jit
def kernel(indices, table):
    flat_idx = indices.reshape(NW, NG, G).astype(jnp.int32)
    mesh = plsc.VectorSubcoreMesh(core_axis_name="c", subcore_axis_name="s")
    run = pl.kernel(
        _gather_body,
        out_type=jax.ShapeDtypeStruct((B, DIM), jnp.float32),
        mesh=mesh,
        scratch_types=[
            pltpu.VMEM((NG, G), jnp.int32),
            pltpu.VMEM((G, DIM), jnp.float32),
            pltpu.SemaphoreType.DMA,
        ],
        compiler_params=pltpu.CompilerParams(use_tc_tiling_on_sc=False),
    )
    out = run(flat_idx, table)
    return out.reshape(BATCH, SEQ, DIM)


# speedup vs baseline: 1.0445x; 1.0445x over previous
import functools

import jax
import jax.numpy as jnp
from jax import lax
from jax.experimental import pallas as pl
from jax.experimental.pallas import tpu as pltpu
from jax.experimental.pallas import tpu_sc as plsc

BATCH = 4096
SEQ = 50
DIM = 64
B = BATCH * SEQ
NC, NS = 2, 16
NW = NC * NS
BPW = B // NW
G = 128
NG = BPW // G


NBUF = 10
NGRP = NG // NBUF


def _gather_body(idx_hbm, table_hbm, out_hbm, idx_v, rows_v, gsem):
    wid = lax.axis_index("s") * NC + lax.axis_index("c")
    base = wid * BPW
    pltpu.sync_copy(idx_hbm.at[wid], idx_v)

    for b in range(NBUF):
        pltpu.async_copy(table_hbm.at[idx_v.at[b]], rows_v.at[b], gsem.at[b])

    def group(g, _):
        for b in range(NBUF):
            j = g * NBUF + b
            pltpu.make_async_copy(
                table_hbm.at[idx_v.at[b]], rows_v.at[b], gsem.at[b]
            ).wait()
            pltpu.sync_copy(rows_v.at[b], out_hbm.at[pl.ds(base + j * G, G)])
            pltpu.async_copy(
                table_hbm.at[idx_v.at[j + NBUF]], rows_v.at[b], gsem.at[b]
            )
        return ()

    lax.fori_loop(0, NGRP - 1, group, ())

    for b in range(NBUF):
        j = (NGRP - 1) * NBUF + b
        pltpu.make_async_copy(
            table_hbm.at[idx_v.at[b]], rows_v.at[b], gsem.at[b]
        ).wait()
        pltpu.sync_copy(rows_v.at[b], out_hbm.at[pl.ds(base + j * G, G)])


@jax.jit
def kernel(indices, table):
    flat_idx = indices.reshape(NW, NG, G).astype(jnp.int32)
    mesh = plsc.VectorSubcoreMesh(core_axis_name="c", subcore_axis_name="s")
    run = pl.kernel(
        _gather_body,
        out_type=jax.ShapeDtypeStruct((B, DIM), jnp.float32),
        mesh=mesh,
        scratch_types=[
            pltpu.VMEM((NG, G), jnp.int32),
            pltpu.VMEM((NBUF, G, DIM), jnp.float32),
            pltpu.SemaphoreType.DMA((NBUF,)),
        ],
        compiler_params=pltpu.CompilerParams(use_tc_tiling_on_sc=False),
    )
    out = run(flat_idx, table)
    return out.reshape(BATCH, SEQ, DIM)


# speedup vs baseline: 1.0612x; 1.0160x over previous
import functools

import jax
import jax.numpy as jnp
from jax import lax
from jax.experimental import pallas as pl
from jax.experimental.pallas import tpu as pltpu
from jax.experimental.pallas import tpu_sc as plsc

BATCH = 4096
SEQ = 50
DIM = 64
B = BATCH * SEQ
NC, NS = 2, 16
NW = NC * NS
BPW = B // NW
G = 128
NG = BPW // G


NBUF = 10
NGRP = NG // NBUF


def _gather_body(idx_hbm, table_hbm, out_hbm, idx_v, rows_v, gsem):
    wid = lax.axis_index("s") * NC + lax.axis_index("c")
    base = wid * BPW
    pltpu.sync_copy(idx_hbm.at[wid], idx_v)

    for b in range(NBUF):
        pltpu.async_copy(table_hbm.at[idx_v.at[b]], rows_v.at[b], gsem.at[b])

    def group(g, _):
        for b in range(NBUF):
            j = g * NBUF + b
            pltpu.make_async_copy(
                table_hbm.at[idx_v.at[b]], rows_v.at[b], gsem.at[b]
            ).wait()
            pltpu.sync_copy(rows_v.at[b], out_hbm.at[pl.ds(base + j * G, G)])
            pltpu.async_copy(
                table_hbm.at[idx_v.at[j + NBUF]], rows_v.at[b], gsem.at[b]
            )
        return ()

    lax.fori_loop(0, NGRP - 1, group, ())

    for b in range(NBUF):
        j = (NGRP - 1) * NBUF + b
        pltpu.make_async_copy(
            table_hbm.at[idx_v.at[b]], rows_v.at[b], gsem.at[b]
        ).wait()
        pltpu.sync_copy(rows_v.at[b], out_hbm.at[pl.ds(base + j * G, G)])


@jax.jit
def kernel(indices, table):
    flat_idx = indices.T.reshape(NW, NG, G).astype(jnp.int32)
    mesh = plsc.VectorSubcoreMesh(core_axis_name="c", subcore_axis_name="s")
    run = pl.kernel(
        _gather_body,
        out_type=jax.ShapeDtypeStruct((B, DIM), jnp.float32),
        mesh=mesh,
        scratch_types=[
            pltpu.VMEM((NG, G), jnp.int32),
            pltpu.VMEM((NBUF, G, DIM), jnp.float32),
            pltpu.SemaphoreType.DMA((NBUF,)),
        ],
        compiler_params=pltpu.CompilerParams(use_tc_tiling_on_sc=False),
    )
    out = run(flat_idx, table)
    return out.reshape(SEQ, BATCH, DIM).transpose(1, 0, 2)
